# baseline (device time: 12707 ns/iter reference)
import jax
import jax.numpy as jnp
from jax import lax
from jax.experimental import pallas as pl
from jax.experimental.pallas import tpu as pltpu

N_DEV = 4
N_HALF = 2
N_ROWCHUNKS = 8


def kernel(x):
    m_per, n = x.shape
    rows_per = m_per // N_ROWCHUNKS
    nc = n // N_HALF

    def body(x_ref, out_ref, acc_ref, rbuf_ref, send_sems, recv_sems):
        g = pl.program_id(0)
        c = g // N_ROWCHUNKS
        r = g % N_ROWCHUNKS
        my_pos = lax.axis_index("i")

        vals = x_ref[:, :]
        mc = jnp.max(vals, axis=0, keepdims=True)
        rows = lax.broadcasted_iota(jnp.int32, (rows_per, nc), 0) + r * rows_per
        masked = jnp.where(vals == mc, rows, N_DEV * m_per)
        mi = jnp.min(masked, axis=0, keepdims=True).astype(jnp.float32)

        def rdmas_for(half):
            out = []
            for d in range(1, N_DEV):
                out.append(
                    pltpu.make_async_remote_copy(
                        src_ref=acc_ref.at[:, half * nc : (half + 1) * nc],
                        dst_ref=rbuf_ref.at[
                            d - 1, :, half * nc : (half + 1) * nc
                        ],
                        send_sem=send_sems.at[half, d - 1],
                        recv_sem=recv_sems.at[half, d - 1],
                        device_id=((my_pos + d) % N_DEV,),
                        device_id_type=pl.DeviceIdType.MESH,
                    )
                )
            return out

        for half in range(N_HALF):
            col = slice(half * nc, (half + 1) * nc)

            @pl.when((c == half) & (r == 0))
            def _():
                acc_ref[0:1, col] = mc
                acc_ref[1:2, col] = mi

            @pl.when((c == half) & (r > 0))
            def _():
                bv = acc_ref[0:1, col]
                take = mc > bv
                acc_ref[0:1, col] = jnp.where(take, mc, bv)
                acc_ref[1:2, col] = jnp.where(take, mi, acc_ref[1:2, col])

            @pl.when((c == half) & (r == N_ROWCHUNKS - 1))
            def _():
                acc_ref[1:2, col] = acc_ref[1:2, col] + (
                    my_pos * m_per
                ).astype(jnp.float32)

                if half == 0:
                    barrier_sem = pltpu.get_barrier_semaphore()
                    for p in range(N_DEV):

                        @pl.when(my_pos != p)
                        def _():
                            pl.semaphore_signal(
                                barrier_sem,
                                inc=1,
                                device_id=(p,),
                                device_id_type=pl.DeviceIdType.MESH,
                            )

                    pl.semaphore_wait(barrier_sem, N_DEV - 1)

                for rdma in rdmas_for(half):
                    rdma.start()

        @pl.when(g == N_HALF * N_ROWCHUNKS - 1)
        def _():
            for half in range(N_HALF):
                for rdma in rdmas_for(half):
                    rdma.wait()

            best_v = acc_ref[0:1, :]
            best_i = acc_ref[1:2, :]
            for k in range(N_DEV - 1):
                v = rbuf_ref[k, 0:1, :]
                i = rbuf_ref[k, 1:2, :]
                take = (v > best_v) | ((v == best_v) & (i < best_i))
                best_v = jnp.where(take, v, best_v)
                best_i = jnp.where(take, i, best_i)
            out_ref[0:1, :] = best_v
            out_ref[1:2, :] = best_i

    return pl.pallas_call(
        body,
        grid=(N_HALF * N_ROWCHUNKS,),
        out_shape=jax.ShapeDtypeStruct((2, n), jnp.float32),
        in_specs=[
            pl.BlockSpec(
                (rows_per, nc),
                lambda g: (g % N_ROWCHUNKS, g // N_ROWCHUNKS),
            )
        ],
        out_specs=pl.BlockSpec((2, n), lambda g: (0, 0)),
        scratch_shapes=[
            pltpu.VMEM((2, n), jnp.float32),
            pltpu.VMEM((N_DEV - 1, 2, n), jnp.float32),
            pltpu.SemaphoreType.DMA((N_HALF, N_DEV - 1)),
            pltpu.SemaphoreType.DMA((N_HALF, N_DEV - 1)),
        ],
        compiler_params=pltpu.CompilerParams(collective_id=0),
    )(x)


# device time: 9084 ns/iter; 1.3988x vs baseline; 1.3988x over previous
import jax
import jax.numpy as jnp
from jax import lax
from jax.experimental import pallas as pl
from jax.experimental.pallas import tpu as pltpu

N_DEV = 4
N_CHUNKS = 8


def kernel(x):
    m_per, n = x.shape
    rp = m_per // N_CHUNKS

    def body(
        x_hbm,
        out_ref,
        vbuf,
        acc_ref,
        rbuf_ref,
        dma_sems,
        send_sems,
        recv_sems,
    ):
        my_pos = lax.axis_index("i")
        barrier_sem = pltpu.get_barrier_semaphore()

        def chunk_copy(i):
            return pltpu.make_async_copy(
                x_hbm.at[pl.ds(i * rp, rp), :],
                vbuf.at[i],
                dma_sems.at[i],
            )

        for i in range(N_CHUNKS):
            chunk_copy(i).start()

        for p in range(N_DEV):

            @pl.when(my_pos != p)
            def _():
                pl.semaphore_signal(
                    barrier_sem,
                    inc=1,
                    device_id=(p,),
                    device_id_type=pl.DeviceIdType.MESH,
                )

        bv = None
        bi = None
        for i in range(N_CHUNKS):
            chunk_copy(i).wait()
            vals = vbuf[i]
            mc = jnp.max(vals, axis=0, keepdims=True)
            base = i * rp + my_pos * m_per
            rows = lax.broadcasted_iota(jnp.int32, (rp, n), 0) + base
            masked = jnp.where(vals == mc, rows, N_DEV * m_per)
            mi = jnp.min(masked, axis=0, keepdims=True).astype(jnp.float32)
            if i == 0:
                bv, bi = mc, mi
            else:
                take = mc > bv
                bv = jnp.where(take, mc, bv)
                bi = jnp.where(take, mi, bi)
        acc_ref[0:1, :] = bv
        acc_ref[1:2, :] = bi

        pl.semaphore_wait(barrier_sem, N_DEV - 1)
        rdmas = []
        for d in range(1, N_DEV):
            rdma = pltpu.make_async_remote_copy(
                src_ref=acc_ref,
                dst_ref=rbuf_ref.at[d - 1],
                send_sem=send_sems.at[d - 1],
                recv_sem=recv_sems.at[d - 1],
                device_id=((my_pos + d) % N_DEV,),
                device_id_type=pl.DeviceIdType.MESH,
            )
            rdma.start()
            rdmas.append(rdma)
        for rdma in rdmas:
            rdma.wait()

        best_v = acc_ref[0:1, :]
        best_i = acc_ref[1:2, :]
        for k in range(N_DEV - 1):
            v = rbuf_ref[k, 0:1, :]
            i = rbuf_ref[k, 1:2, :]
            take = (v > best_v) | ((v == best_v) & (i < best_i))
            best_v = jnp.where(take, v, best_v)
            best_i = jnp.where(take, i, best_i)
        out_ref[0:1, :] = best_v
        out_ref[1:2, :] = best_i

    x = pltpu.with_memory_space_constraint(x, pltpu.HBM)
    return pl.pallas_call(
        body,
        out_shape=jax.ShapeDtypeStruct((2, n), jnp.float32),
        in_specs=[pl.BlockSpec(memory_space=pl.ANY)],
        out_specs=pl.BlockSpec(memory_space=pltpu.VMEM),
        scratch_shapes=[
            pltpu.VMEM((N_CHUNKS, rp, n), jnp.float32),
            pltpu.VMEM((2, n), jnp.float32),
            pltpu.VMEM((N_DEV - 1, 2, n), jnp.float32),
            pltpu.SemaphoreType.DMA((N_CHUNKS,)),
            pltpu.SemaphoreType.DMA((N_DEV - 1,)),
            pltpu.SemaphoreType.DMA((N_DEV - 1,)),
        ],
        compiler_params=pltpu.CompilerParams(collective_id=0),
    )(x)
